# baseline (speedup 1.0000x reference)
import jax
import jax.numpy as jnp
from jax.experimental import pallas as pl
from jax.experimental.pallas import tpu as pltpu

_EPS = 1e-8
_LANES = 128
_ROWS = 8


def _batcher_pairs(n):
    pairs = []
    t = 1
    while (1 << t) < n:
        t += 1
    p = 1 << (t - 1)
    while p > 0:
        q = 1 << (t - 1)
        r = 0
        d = p
        while True:
            for i in range(n - d):
                if (i & p) == r:
                    pairs.append((i, i + d))
            if q == p:
                break
            d = q - p
            r = p
            q >>= 1
        p >>= 1
    return pairs


_SORT_PAIRS = _batcher_pairs(24)
_SIGNS = ((1.0, 1.0), (-1.0, 1.0), (-1.0, -1.0), (1.0, -1.0))


def _corners(x, y, hx, hy, c, s):
    ax, ay = [], []
    for sx, sy in _SIGNS:
        lx = sx * hx
        ly = sy * hy
        ax.append(lx * c - ly * s + x)
        ay.append(lx * s + ly * c + y)
    return ax, ay


def _iou_kernel(inp_ref, tgt_ref, w_ref, out_ref):
    out_ref[...] = w_ref[...]
    return
    x1, y1, z1 = inp_ref[0], inp_ref[1], inp_ref[2]
    dx1, dy1, dz1, h1 = inp_ref[3], inp_ref[4], inp_ref[5], inp_ref[6]
    tch = [tgt_ref[i] for i in range(7)]
    ich = [x1, y1, z1, dx1, dy1, dz1, h1]
    tch = [jnp.where(jnp.isnan(t), i, t) for t, i in zip(tch, ich)]
    x2, y2, z2, dx2, dy2, dz2, h2 = tch

    c1, s1 = jnp.cos(h1), jnp.sin(h1)
    c2, s2 = jnp.cos(h2), jnp.sin(h2)
    hx1, hy1 = dx1 * 0.5, dy1 * 0.5
    hx2, hy2 = dx2 * 0.5, dy2 * 0.5
    ax, ay = _corners(x1, y1, hx1, hy1, c1, s1)
    bx, by = _corners(x2, y2, hx2, hy2, c2, s2)

    px_l, py_l, m_l = [], [], []
    for i in range(4):
        i2 = (i + 1) % 4
        d1x, d1y = ax[i2] - ax[i], ay[i2] - ay[i]
        for j in range(4):
            j2 = (j + 1) % 4
            d2x, d2y = bx[j2] - bx[j], by[j2] - by[j]
            den = d1x * d2y - d1y * d2x
            aden = jnp.abs(den)
            ok = aden >= _EPS
            rinv = 1.0 / jnp.where(ok, den, 1.0)
            wx, wy = bx[j] - ax[i], by[j] - ay[i]
            t = (wx * d2y - wy * d2x) * rinv
            u = (wx * d1y - wy * d1x) * rinv
            m = ok & (t >= 0.0) & (t <= 1.0) & (u >= 0.0) & (u <= 1.0)
            px_l.append(ax[i] + t * d1x)
            py_l.append(ay[i] + t * d1y)
            m_l.append(m)

    for k in range(4):
        rx, ry = ax[k] - x2, ay[k] - y2
        u = rx * c2 + ry * s2
        v = -rx * s2 + ry * c2
        m_l.append((jnp.abs(u) <= hx2 + 1e-6) & (jnp.abs(v) <= hy2 + 1e-6))
        px_l.append(ax[k])
        py_l.append(ay[k])
    for k in range(4):
        rx, ry = bx[k] - x1, by[k] - y1
        u = rx * c1 + ry * s1
        v = -rx * s1 + ry * c1
        m_l.append((jnp.abs(u) <= hx1 + 1e-6) & (jnp.abs(v) <= hy1 + 1e-6))
        px_l.append(bx[k])
        py_l.append(by[k])

    px_l = [jnp.where(m, p, 0.0) for m, p in zip(m_l, px_l)]
    py_l = [jnp.where(m, p, 0.0) for m, p in zip(m_l, py_l)]

    sx = px_l[0]
    sy = py_l[0]
    cnt = m_l[0].astype(jnp.float32)
    for k in range(1, 24):
        sx = sx + px_l[k]
        sy = sy + py_l[k]
        cnt = cnt + m_l[k].astype(jnp.float32)
    rcnt = 1.0 / jnp.maximum(cnt, 1.0)
    cx = sx * rcnt
    cy = sy * rcnt

    K, X, Y = [], [], []
    for k in range(24):
        rx = px_l[k] - cx
        ry = py_l[k] - cy
        ad = jnp.maximum(jnp.abs(rx) + jnp.abs(ry), 1e-30)
        base = 1.0 - rx / ad
        key = jnp.where(ry < 0.0, -base, base)
        K.append(jnp.where(m_l[k], key, 1e6))
        X.append(rx)
        Y.append(ry)

    for i, j in _SORT_PAIRS:
        ki, kj = K[i], K[j]
        sw = ki > kj
        K[i] = jnp.minimum(ki, kj)
        K[j] = jnp.maximum(ki, kj)
        xi, xj = X[i], X[j]
        X[i] = jnp.where(sw, xj, xi)
        X[j] = jnp.where(sw, xi, xj)
        yi, yj = Y[i], Y[j]
        Y[i] = jnp.where(sw, yj, yi)
        Y[j] = jnp.where(sw, yi, yj)

    fx, fy = X[0], Y[0]
    for k in range(1, 24):
        inv = K[k] >= 5e5
        X[k] = jnp.where(inv, fx, X[k])
        Y[k] = jnp.where(inv, fy, Y[k])

    acc = X[23] * Y[0] - Y[23] * X[0]
    for k in range(23):
        acc = acc + X[k] * Y[k + 1] - Y[k] * X[k + 1]
    area = 0.5 * jnp.abs(acc)

    zmax = jnp.minimum(z1 + dz1 * 0.5, z2 + dz2 * 0.5)
    zmin = jnp.maximum(z1 - dz1 * 0.5, z2 - dz2 * 0.5)
    inter = area * jnp.maximum(zmax - zmin, 0.0)
    vol1 = dx1 * dy1 * dz1
    vol2 = dx2 * dy2 * dz2
    iou = inter / jnp.maximum(vol1 + vol2 - inter, 1e-7)
    out_ref[...] = (1.0 - iou) * w_ref[...]


def kernel(input, target, weights):
    n = input.shape[0]
    blk = _ROWS * _LANES
    rows = n // _LANES
    inp = input.reshape(rows, _LANES, 7)
    tgt = target.reshape(rows, _LANES, 7)
    w = weights.reshape(rows, _LANES)
    out = pl.pallas_call(
        _iou_kernel,
        grid=(n // blk,),
        in_specs=[
            pl.BlockSpec((_ROWS, _LANES, 7), lambda i: (i, 0, 0)),
            pl.BlockSpec((_ROWS, _LANES, 7), lambda i: (i, 0, 0)),
            pl.BlockSpec((_ROWS, _LANES), lambda i: (i, 0)),
        ],
        out_specs=pl.BlockSpec((_ROWS, _LANES), lambda i: (i, 0)),
        out_shape=jax.ShapeDtypeStruct((rows, _LANES), jnp.float32),
        compiler_params=pltpu.CompilerParams(
            dimension_semantics=("parallel",)
        ),
    )(inp, tgt, w)
    return out.reshape(n)


# speedup vs baseline: 1.9651x; 1.9651x over previous
import jax
import jax.numpy as jnp
from jax.experimental import pallas as pl
from jax.experimental.pallas import tpu as pltpu

_EPS = 1e-8
_LANES = 128
_ROWS = 8
_BROWS = 128


def _batcher_pairs(n):
    pairs = []
    t = 1
    while (1 << t) < n:
        t += 1
    p = 1 << (t - 1)
    while p > 0:
        q = 1 << (t - 1)
        r = 0
        d = p
        while True:
            for i in range(n - d):
                if (i & p) == r:
                    pairs.append((i, i + d))
            if q == p:
                break
            d = q - p
            r = p
            q >>= 1
        p >>= 1
    return pairs


_SORT_PAIRS = _batcher_pairs(24)
_SIGNS = ((1.0, 1.0), (-1.0, 1.0), (-1.0, -1.0), (1.0, -1.0))


def _corners(x, y, hx, hy, c, s):
    ax, ay = [], []
    for sx, sy in _SIGNS:
        lx = sx * hx
        ly = sy * hy
        ax.append(lx * c - ly * s + x)
        ay.append(lx * s + ly * c + y)
    return ax, ay


def _iou_kernel(inp_ref, tgt_ref, w_ref, out_ref):
    for k in range(_BROWS // _ROWS):
        sl = slice(k * _ROWS, (k + 1) * _ROWS)
        _iou_chunk(inp_ref, tgt_ref, w_ref, out_ref, sl)


def _iou_chunk(inp_ref, tgt_ref, w_ref, out_ref, sl):
    x1, y1, z1 = inp_ref[0, sl], inp_ref[1, sl], inp_ref[2, sl]
    dx1, dy1, dz1 = inp_ref[3, sl], inp_ref[4, sl], inp_ref[5, sl]
    h1 = inp_ref[6, sl]
    tch = [tgt_ref[i, sl] for i in range(7)]
    ich = [x1, y1, z1, dx1, dy1, dz1, h1]
    tch = [jnp.where(jnp.isnan(t), i, t) for t, i in zip(tch, ich)]
    x2, y2, z2, dx2, dy2, dz2, h2 = tch

    c1, s1 = jnp.cos(h1), jnp.sin(h1)
    c2, s2 = jnp.cos(h2), jnp.sin(h2)
    hx1, hy1 = dx1 * 0.5, dy1 * 0.5
    hx2, hy2 = dx2 * 0.5, dy2 * 0.5
    ax, ay = _corners(x1, y1, hx1, hy1, c1, s1)
    bx, by = _corners(x2, y2, hx2, hy2, c2, s2)

    px_l, py_l, m_l = [], [], []
    for i in range(4):
        i2 = (i + 1) % 4
        d1x, d1y = ax[i2] - ax[i], ay[i2] - ay[i]
        for j in range(4):
            j2 = (j + 1) % 4
            d2x, d2y = bx[j2] - bx[j], by[j2] - by[j]
            den = d1x * d2y - d1y * d2x
            aden = jnp.abs(den)
            ok = aden >= _EPS
            rinv = 1.0 / jnp.where(ok, den, 1.0)
            wx, wy = bx[j] - ax[i], by[j] - ay[i]
            t = (wx * d2y - wy * d2x) * rinv
            u = (wx * d1y - wy * d1x) * rinv
            m = ok & (t >= 0.0) & (t <= 1.0) & (u >= 0.0) & (u <= 1.0)
            px_l.append(ax[i] + t * d1x)
            py_l.append(ay[i] + t * d1y)
            m_l.append(m)

    for k in range(4):
        rx, ry = ax[k] - x2, ay[k] - y2
        u = rx * c2 + ry * s2
        v = -rx * s2 + ry * c2
        m_l.append((jnp.abs(u) <= hx2 + 1e-6) & (jnp.abs(v) <= hy2 + 1e-6))
        px_l.append(ax[k])
        py_l.append(ay[k])
    for k in range(4):
        rx, ry = bx[k] - x1, by[k] - y1
        u = rx * c1 + ry * s1
        v = -rx * s1 + ry * c1
        m_l.append((jnp.abs(u) <= hx1 + 1e-6) & (jnp.abs(v) <= hy1 + 1e-6))
        px_l.append(bx[k])
        py_l.append(by[k])

    px_l = [jnp.where(m, p, 0.0) for m, p in zip(m_l, px_l)]
    py_l = [jnp.where(m, p, 0.0) for m, p in zip(m_l, py_l)]

    sx = px_l[0]
    sy = py_l[0]
    cnt = m_l[0].astype(jnp.float32)
    for k in range(1, 24):
        sx = sx + px_l[k]
        sy = sy + py_l[k]
        cnt = cnt + m_l[k].astype(jnp.float32)
    rcnt = 1.0 / jnp.maximum(cnt, 1.0)
    cx = sx * rcnt
    cy = sy * rcnt

    K, X, Y = [], [], []
    for k in range(24):
        rx = px_l[k] - cx
        ry = py_l[k] - cy
        ad = jnp.maximum(jnp.abs(rx) + jnp.abs(ry), 1e-30)
        base = 1.0 - rx / ad
        key = jnp.where(ry < 0.0, -base, base)
        K.append(jnp.where(m_l[k], key, 1e6))
        X.append(rx)
        Y.append(ry)

    for i, j in _SORT_PAIRS:
        ki, kj = K[i], K[j]
        sw = ki > kj
        K[i] = jnp.minimum(ki, kj)
        K[j] = jnp.maximum(ki, kj)
        xi, xj = X[i], X[j]
        X[i] = jnp.where(sw, xj, xi)
        X[j] = jnp.where(sw, xi, xj)
        yi, yj = Y[i], Y[j]
        Y[i] = jnp.where(sw, yj, yi)
        Y[j] = jnp.where(sw, yi, yj)

    fx, fy = X[0], Y[0]
    for k in range(1, 24):
        inv = K[k] >= 5e5
        X[k] = jnp.where(inv, fx, X[k])
        Y[k] = jnp.where(inv, fy, Y[k])

    acc = X[23] * Y[0] - Y[23] * X[0]
    for k in range(23):
        acc = acc + X[k] * Y[k + 1] - Y[k] * X[k + 1]
    area = 0.5 * jnp.abs(acc)

    zmax = jnp.minimum(z1 + dz1 * 0.5, z2 + dz2 * 0.5)
    zmin = jnp.maximum(z1 - dz1 * 0.5, z2 - dz2 * 0.5)
    inter = area * jnp.maximum(zmax - zmin, 0.0)
    vol1 = dx1 * dy1 * dz1
    vol2 = dx2 * dy2 * dz2
    iou = inter / jnp.maximum(vol1 + vol2 - inter, 1e-7)
    out_ref[sl] = (1.0 - iou) * w_ref[sl]


def kernel(input, target, weights):
    n = input.shape[0]
    blk = _BROWS * _LANES
    rows = n // _LANES
    inp = input.T.reshape(7, rows, _LANES)
    tgt = target.T.reshape(7, rows, _LANES)
    w = weights.reshape(rows, _LANES)
    out = pl.pallas_call(
        _iou_kernel,
        grid=(n // blk,),
        in_specs=[
            pl.BlockSpec((7, _BROWS, _LANES), lambda i: (0, i, 0)),
            pl.BlockSpec((7, _BROWS, _LANES), lambda i: (0, i, 0)),
            pl.BlockSpec((_BROWS, _LANES), lambda i: (i, 0)),
        ],
        out_specs=pl.BlockSpec((_BROWS, _LANES), lambda i: (i, 0)),
        out_shape=jax.ShapeDtypeStruct((rows, _LANES), jnp.float32),
        compiler_params=pltpu.CompilerParams(
            dimension_semantics=("parallel",)
        ),
    )(inp, tgt, w)
    return out.reshape(n)
